# baseline (speedup 1.0000x reference)
import functools

import jax
import jax.numpy as jnp
from jax import lax
from jax.experimental import pallas as pl
from jax.experimental.pallas import tpu as pltpu
from jax.experimental.pallas import tpu_sc as plsc

_NUM_CORES = 2
_NUM_SUBCORES = 16
_NUM_WORKERS = _NUM_CORES * _NUM_SUBCORES
_BLK = 512


def _transpose_body(in_ref, out_ref):
    for s in range(_BLK // 128):
        d = s * 32
        out_ref[:, d:d + 32] = in_ref[:, s * 128:(s + 1) * 128].T


@functools.cache
def _make_transpose(num_rows, dim):
    n_blocks = (num_rows + _BLK - 1) // _BLK

    return pl.pallas_call(
        _transpose_body,
        grid=(n_blocks,),
        in_specs=[pl.BlockSpec((dim, _BLK), lambda i: (0, i))],
        out_specs=pl.BlockSpec((128, 128), lambda i: (i, 0)),
        out_shape=jax.ShapeDtypeStruct((n_blocks * 128, 128), jnp.float32),
    )


@functools.cache
def _make_gather(num_rows, dim, rows):
    assert rows % (8 * _NUM_WORKERS) == 0
    r_per_w = rows // _NUM_WORKERS
    mesh = plsc.VectorSubcoreMesh(
        core_axis_name="c",
        subcore_axis_name="s",
        num_cores=_NUM_CORES,
        num_subcores=_NUM_SUBCORES,
    )

    @functools.partial(
        pl.kernel,
        mesh=mesh,
        out_type=jax.ShapeDtypeStruct((rows, dim), jnp.float32),
        scratch_types=[
            pltpu.VMEM((r_per_w,), jnp.int32),
            pltpu.VMEM((r_per_w, dim), jnp.float32),
            pltpu.SemaphoreType.DMA,
        ],
        compiler_params=pltpu.CompilerParams(use_tc_tiling_on_sc=False),
    )
    def gather(idx_hbm, table_hbm, out_hbm, idx_v, rows_v, sem):
        wid = lax.axis_index("s") * _NUM_CORES + lax.axis_index("c")
        base = wid * r_per_w
        pltpu.sync_copy(idx_hbm.at[pl.ds(base, r_per_w)], idx_v)
        pltpu.async_copy(table_hbm.at[idx_v], rows_v, sem).wait()
        pltpu.sync_copy(rows_v, out_hbm.at[pl.ds(base, r_per_w)])

    return gather


def _kernel_impl(inputs, table):
    batch, n_fields = inputs.shape
    num_rows, dim = table.shape
    idx = inputs.reshape(-1).astype(jnp.int32)
    groups = _BLK // 128
    idx = (
        groups * (128 * (idx // _BLK) + idx % 128) + (idx // 128) % groups
    )
    lin128 = _make_transpose(num_rows, dim)(table.T)
    lin = lin128.reshape(lin128.shape[0] * (128 // dim), dim)
    out = _make_gather(lin.shape[0], dim, batch * n_fields)(idx, lin)
    return out.reshape(batch, n_fields, dim)


kernel = jax.jit(_kernel_impl)


# speedup vs baseline: 3.5975x; 3.5975x over previous
import functools

import jax
import jax.numpy as jnp
from jax import lax
from jax.experimental import pallas as pl
from jax.experimental.pallas import tpu as pltpu
from jax.experimental.pallas import tpu_sc as plsc

_NUM_CORES = 2
_NUM_SUBCORES = 16
_NUM_WORKERS = _NUM_CORES * _NUM_SUBCORES
_LANES = 16
_SB = 512


@functools.cache
def _mesh():
    return plsc.VectorSubcoreMesh(
        core_axis_name="c",
        subcore_axis_name="s",
        num_cores=_NUM_CORES,
        num_subcores=_NUM_SUBCORES,
    )


@functools.cache
def _make_detile(num_rows, dim):
    full_blocks = num_rows // _SB
    tail = num_rows % _SB
    per_w = full_blocks // _NUM_WORKERS
    extra = full_blocks % _NUM_WORKERS
    block_elems = _SB * dim

    @functools.partial(
        pl.kernel,
        mesh=_mesh(),
        out_type=jax.ShapeDtypeStruct((num_rows * dim,), jnp.float32),
        scratch_types=[
            pltpu.VMEM((dim, _SB), jnp.float32),
            pltpu.VMEM((dim, _SB), jnp.float32),
            pltpu.VMEM((block_elems,), jnp.float32),
            pltpu.VMEM((block_elems,), jnp.float32),
            pltpu.VMEM((max(tail, 1) * dim,), jnp.float32),
            pltpu.SemaphoreType.DMA,
            pltpu.SemaphoreType.DMA,
            pltpu.SemaphoreType.DMA,
            pltpu.SemaphoreType.DMA,
        ],
        compiler_params=pltpu.CompilerParams(
            use_tc_tiling_on_sc=True, needs_layout_passes=False
        ),
    )
    def detile(tab_t, tail1d, lin, in0, in1, st0, st1, tailv,
               sin0, sin1, sout0, sout1):
        wid = lax.axis_index("s") * _NUM_CORES + lax.axis_index("c")
        n_blocks = jnp.where(wid < extra, per_w + 1, per_w)
        start = wid * per_w + jnp.minimum(wid, extra)
        lanes = lax.iota(jnp.int32, _LANES)
        lanes_d = lanes * dim
        jvs = [(lanes + j0) % dim for j0 in range(dim)]
        bufs = ((in0, st0, sin0, sout0), (in1, st1, sin1, sout1))

        def src(i):
            return tab_t.at[:, pl.ds((start + i) * _SB, _SB)]

        def dst(i):
            return lin.at[pl.ds((start + i) * block_elems, block_elems)]

        pltpu.async_copy(src(0), in0, sin0)

        @pl.when(n_blocks > 1)
        def _():
            pltpu.async_copy(src(1), in1, sin1)

        def step(i, p):
            inb, st, sin, sout = bufs[p]

            @pl.when(i >= 2)
            def _():
                pltpu.make_async_copy(st, dst(i - 2), sout).wait()

            pltpu.make_async_copy(src(i), inb, sin).wait()

            def inner(cc, carry):
                c0 = cc * _LANES
                cv = lanes + c0
                base = lanes_d + c0 * dim
                for j0 in range(dim):
                    x = plsc.load_gather(inb, [jvs[j0], cv])
                    plsc.store_scatter(st, [base + jvs[j0]], x)
                return carry

            lax.fori_loop(0, _SB // _LANES, inner, 0)
            pltpu.async_copy(st, dst(i), sout)

            @pl.when(i + 2 < n_blocks)
            def _():
                pltpu.async_copy(src(i + 2), inb, sin)

        def body(k, carry):
            step(2 * k, 0)

            @pl.when(2 * k + 1 < n_blocks)
            def _():
                step(2 * k + 1, 1)

            return carry

        lax.fori_loop(0, (n_blocks + 1) // 2, body, 0)

        @pl.when(n_blocks >= 2)
        def _():
            pltpu.make_async_copy(st0, dst(n_blocks - 2), sout0).wait()
            pltpu.make_async_copy(st1, dst(n_blocks - 2), sout1).wait()

        @pl.when(n_blocks == 1)
        def _():
            pltpu.make_async_copy(st0, dst(0), sout0).wait()

        if tail:
            @pl.when(wid == _NUM_WORKERS - 1)
            def _():
                pltpu.sync_copy(tail1d, tailv)
                pltpu.sync_copy(
                    tailv, lin.at[pl.ds(full_blocks * block_elems, tail * dim)]
                )

    return detile


@functools.cache
def _make_gather(num_rows, dim, rows):
    assert rows % (8 * _NUM_WORKERS) == 0
    r_per_w = rows // _NUM_WORKERS

    @functools.partial(
        pl.kernel,
        mesh=_mesh(),
        out_type=jax.ShapeDtypeStruct((rows, dim), jnp.float32),
        scratch_types=[
            pltpu.VMEM((r_per_w,), jnp.int32),
            pltpu.VMEM((r_per_w, dim), jnp.float32),
            pltpu.SemaphoreType.DMA,
        ],
        compiler_params=pltpu.CompilerParams(use_tc_tiling_on_sc=False),
    )
    def gather(idx_hbm, table_hbm, out_hbm, idx_v, rows_v, sem):
        wid = lax.axis_index("s") * _NUM_CORES + lax.axis_index("c")
        base = wid * r_per_w
        pltpu.sync_copy(idx_hbm.at[pl.ds(base, r_per_w)], idx_v)
        pltpu.async_copy(table_hbm.at[idx_v], rows_v, sem).wait()
        pltpu.sync_copy(rows_v, out_hbm.at[pl.ds(base, r_per_w)])

    return gather


def _kernel_impl(inputs, table):
    batch, n_fields = inputs.shape
    num_rows, dim = table.shape
    idx = inputs.reshape(-1).astype(jnp.int32)
    tail_start = (num_rows // _SB) * _SB
    tail1d = lax.slice(table, (tail_start, 0), (num_rows, dim)).reshape(-1)
    lin = _make_detile(num_rows, dim)(table.T, tail1d)
    out = _make_gather(num_rows, dim, batch * n_fields)(
        idx, lin.reshape(num_rows, dim)
    )
    return out.reshape(batch, n_fields, dim)


kernel = jax.jit(_kernel_impl)


# speedup vs baseline: 6.3083x; 1.7535x over previous
import functools

import jax
import jax.numpy as jnp
from jax import lax
from jax.experimental import pallas as pl
from jax.experimental.pallas import tpu as pltpu
from jax.experimental.pallas import tpu_sc as plsc

_NUM_CORES = 2
_NUM_SUBCORES = 16
_NUM_WORKERS = _NUM_CORES * _NUM_SUBCORES
_LANES = 16
_SB = 512


@functools.cache
def _mesh():
    return plsc.VectorSubcoreMesh(
        core_axis_name="c",
        subcore_axis_name="s",
        num_cores=_NUM_CORES,
        num_subcores=_NUM_SUBCORES,
    )


@functools.cache
def _make_detile(num_rows, dim):
    full_blocks = num_rows // _SB
    tail = num_rows % _SB
    per_w = full_blocks // _NUM_WORKERS
    extra = full_blocks % _NUM_WORKERS
    block_elems = _SB * dim

    @functools.partial(
        pl.kernel,
        mesh=_mesh(),
        out_type=jax.ShapeDtypeStruct((num_rows * dim,), jnp.float32),
        scratch_types=[
            pltpu.VMEM((dim, _SB), jnp.float32),
            pltpu.VMEM((dim, _SB), jnp.float32),
            pltpu.VMEM((block_elems,), jnp.float32),
            pltpu.VMEM((block_elems,), jnp.float32),
            pltpu.VMEM((max(tail, 1) * dim,), jnp.float32),
            pltpu.SemaphoreType.DMA,
            pltpu.SemaphoreType.DMA,
            pltpu.SemaphoreType.DMA,
            pltpu.SemaphoreType.DMA,
        ],
        compiler_params=pltpu.CompilerParams(
            use_tc_tiling_on_sc=True, needs_layout_passes=False
        ),
    )
    def detile(tab_t, tail1d, lin, in0, in1, st0, st1, tailv,
               sin0, sin1, sout0, sout1):
        wid = lax.axis_index("s") * _NUM_CORES + lax.axis_index("c")
        n_blocks = jnp.where(wid < extra, per_w + 1, per_w)
        start = wid * per_w + jnp.minimum(wid, extra)
        lanes = lax.iota(jnp.int32, _LANES)
        lanes_d = lanes * dim
        jvs = [(lanes + j0) % dim for j0 in range(dim)]
        bufs = ((in0, st0, sin0, sout0), (in1, st1, sin1, sout1))

        def src(i):
            return tab_t.at[:, pl.ds((start + i) * _SB, _SB)]

        def dst(i):
            return lin.at[pl.ds((start + i) * block_elems, block_elems)]

        pltpu.async_copy(src(0), in0, sin0)

        @pl.when(n_blocks > 1)
        def _():
            pltpu.async_copy(src(1), in1, sin1)

        def step(i, p):
            inb, st, sin, sout = bufs[p]

            @pl.when(i >= 2)
            def _():
                pltpu.make_async_copy(st, dst(i - 2), sout).wait()

            pltpu.make_async_copy(src(i), inb, sin).wait()

            @plsc.parallel_loop(0, _SB // _LANES, unroll=2)
            def inner(cc):
                c0 = cc * _LANES
                cv = lanes + c0
                base = lanes_d + c0 * dim
                for j0 in range(dim):
                    x = plsc.load_gather(inb, [jvs[j0], cv])
                    plsc.store_scatter(st, [base + jvs[j0]], x)
            pltpu.async_copy(st, dst(i), sout)

            @pl.when(i + 2 < n_blocks)
            def _():
                pltpu.async_copy(src(i + 2), inb, sin)

        def body(k, carry):
            step(2 * k, 0)

            @pl.when(2 * k + 1 < n_blocks)
            def _():
                step(2 * k + 1, 1)

            return carry

        lax.fori_loop(0, (n_blocks + 1) // 2, body, 0)

        @pl.when(n_blocks >= 2)
        def _():
            pltpu.make_async_copy(st0, dst(n_blocks - 2), sout0).wait()
            pltpu.make_async_copy(st1, dst(n_blocks - 2), sout1).wait()

        @pl.when(n_blocks == 1)
        def _():
            pltpu.make_async_copy(st0, dst(0), sout0).wait()

        if tail:
            @pl.when(wid == _NUM_WORKERS - 1)
            def _():
                pltpu.sync_copy(tail1d, tailv)
                pltpu.sync_copy(
                    tailv, lin.at[pl.ds(full_blocks * block_elems, tail * dim)]
                )

    return detile


@functools.cache
def _make_gather(num_rows, dim, rows):
    assert rows % (8 * _NUM_WORKERS) == 0
    r_per_w = rows // _NUM_WORKERS

    @functools.partial(
        pl.kernel,
        mesh=_mesh(),
        out_type=jax.ShapeDtypeStruct((rows, dim), jnp.float32),
        scratch_types=[
            pltpu.VMEM((r_per_w,), jnp.int32),
            pltpu.VMEM((r_per_w, dim), jnp.float32),
            pltpu.SemaphoreType.DMA,
        ],
        compiler_params=pltpu.CompilerParams(use_tc_tiling_on_sc=False),
    )
    def gather(idx_hbm, table_hbm, out_hbm, idx_v, rows_v, sem):
        wid = lax.axis_index("s") * _NUM_CORES + lax.axis_index("c")
        base = wid * r_per_w
        pltpu.sync_copy(idx_hbm.at[pl.ds(base, r_per_w)], idx_v)
        pltpu.async_copy(table_hbm.at[idx_v], rows_v, sem).wait()
        pltpu.sync_copy(rows_v, out_hbm.at[pl.ds(base, r_per_w)])

    return gather


def _kernel_impl(inputs, table):
    batch, n_fields = inputs.shape
    num_rows, dim = table.shape
    idx = inputs.reshape(-1).astype(jnp.int32)
    tail_start = (num_rows // _SB) * _SB
    tail1d = lax.slice(table, (tail_start, 0), (num_rows, dim)).reshape(-1)
    lin = _make_detile(num_rows, dim)(table.T, tail1d)
    out = _make_gather(num_rows, dim, batch * n_fields)(
        idx, lin.reshape(num_rows, dim)
    )
    return out.reshape(batch, n_fields, dim)


kernel = jax.jit(_kernel_impl)


# speedup vs baseline: 6.5051x; 1.0312x over previous
import functools

import jax
import jax.numpy as jnp
from jax import lax
from jax.experimental import pallas as pl
from jax.experimental.pallas import tpu as pltpu
from jax.experimental.pallas import tpu_sc as plsc

_NUM_CORES = 2
_NUM_SUBCORES = 16
_NUM_WORKERS = _NUM_CORES * _NUM_SUBCORES
_LANES = 16
_SB = 512


@functools.cache
def _mesh():
    return plsc.VectorSubcoreMesh(
        core_axis_name="c",
        subcore_axis_name="s",
        num_cores=_NUM_CORES,
        num_subcores=_NUM_SUBCORES,
    )


@functools.cache
def _make_detile(num_rows, dim):
    full_blocks = num_rows // _SB
    tail = num_rows % _SB
    per_w = full_blocks // _NUM_WORKERS
    extra = full_blocks % _NUM_WORKERS
    block_elems = _SB * dim

    @functools.partial(
        pl.kernel,
        mesh=_mesh(),
        out_type=jax.ShapeDtypeStruct((num_rows * dim,), jnp.float32),
        scratch_types=[
            pltpu.VMEM((dim, _SB), jnp.float32),
            pltpu.VMEM((dim, _SB), jnp.float32),
            pltpu.VMEM((block_elems,), jnp.float32),
            pltpu.VMEM((block_elems,), jnp.float32),
            pltpu.VMEM((max(tail, 1) * dim,), jnp.float32),
            pltpu.SemaphoreType.DMA,
            pltpu.SemaphoreType.DMA,
            pltpu.SemaphoreType.DMA,
            pltpu.SemaphoreType.DMA,
        ],
        compiler_params=pltpu.CompilerParams(
            use_tc_tiling_on_sc=True, needs_layout_passes=False
        ),
    )
    def detile(tab_t, tail1d, lin, in0, in1, st0, st1, tailv,
               sin0, sin1, sout0, sout1):
        wid = lax.axis_index("s") * _NUM_CORES + lax.axis_index("c")
        n_blocks = jnp.where(wid < extra, per_w + 1, per_w)
        start = wid * per_w + jnp.minimum(wid, extra)
        lanes = lax.iota(jnp.int32, _LANES)
        lanes_d = lanes * dim
        jvs = [(lanes + j0) % dim for j0 in range(dim)]
        bufs = ((in0, st0, sin0, sout0), (in1, st1, sin1, sout1))

        def src(i):
            return tab_t.at[:, pl.ds((start + i) * _SB, _SB)]

        def dst(i):
            return lin.at[pl.ds((start + i) * block_elems, block_elems)]

        pltpu.async_copy(src(0), in0, sin0)

        @pl.when(n_blocks > 1)
        def _():
            pltpu.async_copy(src(1), in1, sin1)

        def step(i, p):
            inb, st, sin, sout = bufs[p]

            @pl.when(i >= 2)
            def _():
                pltpu.make_async_copy(st, dst(i - 2), sout).wait()

            pltpu.make_async_copy(src(i), inb, sin).wait()

            @plsc.parallel_loop(0, _SB // _LANES, unroll=4)
            def inner(cc):
                c0 = cc * _LANES
                cv = lanes + c0
                base = lanes_d + c0 * dim
                for j0 in range(dim):
                    x = plsc.load_gather(inb, [jvs[j0], cv])
                    plsc.store_scatter(st, [base + jvs[j0]], x)
            pltpu.async_copy(st, dst(i), sout)

            @pl.when(i + 2 < n_blocks)
            def _():
                pltpu.async_copy(src(i + 2), inb, sin)

        def body(k, carry):
            step(2 * k, 0)

            @pl.when(2 * k + 1 < n_blocks)
            def _():
                step(2 * k + 1, 1)

            return carry

        lax.fori_loop(0, (n_blocks + 1) // 2, body, 0)

        @pl.when(n_blocks >= 2)
        def _():
            pltpu.make_async_copy(st0, dst(n_blocks - 2), sout0).wait()
            pltpu.make_async_copy(st1, dst(n_blocks - 2), sout1).wait()

        @pl.when(n_blocks == 1)
        def _():
            pltpu.make_async_copy(st0, dst(0), sout0).wait()

        if tail:
            @pl.when(wid == _NUM_WORKERS - 1)
            def _():
                pltpu.sync_copy(tail1d, tailv)
                pltpu.sync_copy(
                    tailv, lin.at[pl.ds(full_blocks * block_elems, tail * dim)]
                )

    return detile


@functools.cache
def _make_gather(num_rows, dim, rows):
    assert rows % (8 * _NUM_WORKERS) == 0
    r_per_w = rows // _NUM_WORKERS

    @functools.partial(
        pl.kernel,
        mesh=_mesh(),
        out_type=jax.ShapeDtypeStruct((rows, dim), jnp.float32),
        scratch_types=[
            pltpu.VMEM((r_per_w,), jnp.int32),
            pltpu.VMEM((r_per_w, dim), jnp.float32),
            pltpu.SemaphoreType.DMA,
        ],
        compiler_params=pltpu.CompilerParams(use_tc_tiling_on_sc=False),
    )
    def gather(idx_hbm, table_hbm, out_hbm, idx_v, rows_v, sem):
        wid = lax.axis_index("s") * _NUM_CORES + lax.axis_index("c")
        base = wid * r_per_w
        pltpu.sync_copy(idx_hbm.at[pl.ds(base, r_per_w)], idx_v)
        pltpu.async_copy(table_hbm.at[idx_v], rows_v, sem).wait()
        pltpu.sync_copy(rows_v, out_hbm.at[pl.ds(base, r_per_w)])

    return gather


def _kernel_impl(inputs, table):
    batch, n_fields = inputs.shape
    num_rows, dim = table.shape
    idx = inputs.reshape(-1).astype(jnp.int32)
    tail_start = (num_rows // _SB) * _SB
    tail1d = lax.slice(table, (tail_start, 0), (num_rows, dim)).reshape(-1)
    lin = _make_detile(num_rows, dim)(table.T, tail1d)
    out = _make_gather(num_rows, dim, batch * n_fields)(
        idx, lin.reshape(num_rows, dim)
    )
    return out.reshape(batch, n_fields, dim)


kernel = jax.jit(_kernel_impl)


# speedup vs baseline: 8.6737x; 1.3334x over previous
import functools

import jax
import jax.numpy as jnp
from jax import lax
from jax.experimental import pallas as pl
from jax.experimental.pallas import tpu as pltpu
from jax.experimental.pallas import tpu_sc as plsc

_NUM_CORES = 2
_NUM_SUBCORES = 16
_NUM_WORKERS = _NUM_CORES * _NUM_SUBCORES
_LANES = 16
_SB = 512


@functools.cache
def _mesh():
    return plsc.VectorSubcoreMesh(
        core_axis_name="c",
        subcore_axis_name="s",
        num_cores=_NUM_CORES,
        num_subcores=_NUM_SUBCORES,
    )


@functools.cache
def _make_detile(num_rows, dim):
    full_blocks = num_rows // _SB
    tail = num_rows % _SB
    per_w = full_blocks // _NUM_WORKERS
    extra = full_blocks % _NUM_WORKERS
    block_elems = _SB * dim

    @functools.partial(
        pl.kernel,
        mesh=_mesh(),
        out_type=jax.ShapeDtypeStruct((num_rows * dim,), jnp.float32),
        scratch_types=[
            pltpu.VMEM((dim, _SB), jnp.float32),
            pltpu.VMEM((dim, _SB), jnp.float32),
            pltpu.VMEM((block_elems,), jnp.float32),
            pltpu.VMEM((block_elems,), jnp.float32),
            pltpu.VMEM((max(tail, 1) * dim,), jnp.float32),
            pltpu.SemaphoreType.DMA,
            pltpu.SemaphoreType.DMA,
            pltpu.SemaphoreType.DMA,
            pltpu.SemaphoreType.DMA,
        ],
        compiler_params=pltpu.CompilerParams(
            use_tc_tiling_on_sc=True, needs_layout_passes=False
        ),
    )
    def detile(tab_t, tail1d, lin, in0, in1, st0, st1, tailv,
               sin0, sin1, sout0, sout1):
        wid = lax.axis_index("s") * _NUM_CORES + lax.axis_index("c")
        n_blocks = jnp.where(wid < extra, per_w + 1, per_w)
        start = wid * per_w + jnp.minimum(wid, extra)
        lanes = lax.iota(jnp.int32, _LANES)
        lanes_d = lanes * dim
        jvs = [(lanes + j0) % dim for j0 in range(dim)]
        bufs = ((in0, st0, sin0, sout0), (in1, st1, sin1, sout1))

        def src(i):
            return tab_t.at[:, pl.ds((start + i) * _SB, _SB)]

        def dst(i):
            return lin.at[pl.ds((start + i) * block_elems, block_elems)]

        pltpu.async_copy(src(0), in0, sin0)

        @pl.when(n_blocks > 1)
        def _():
            pltpu.async_copy(src(1), in1, sin1)

        def step(i, p):
            inb, st, sin, sout = bufs[p]

            @pl.when(i >= 2)
            def _():
                pltpu.make_async_copy(st, dst(i - 2), sout).wait()

            pltpu.make_async_copy(src(i), inb, sin).wait()

            @plsc.parallel_loop(0, _SB // _LANES, unroll=4)
            def inner(cc):
                c0 = cc * _LANES
                cv = lanes + c0
                base = lanes_d + c0 * dim
                for j0 in range(dim):
                    x = plsc.load_gather(inb, [jvs[j0], cv])
                    plsc.store_scatter(st, [base + jvs[j0]], x)
            pltpu.async_copy(st, dst(i), sout)

            @pl.when(i + 2 < n_blocks)
            def _():
                pltpu.async_copy(src(i + 2), inb, sin)

        def body(k, carry):
            step(2 * k, 0)

            @pl.when(2 * k + 1 < n_blocks)
            def _():
                step(2 * k + 1, 1)

            return carry

        lax.fori_loop(0, (n_blocks + 1) // 2, body, 0)

        @pl.when(n_blocks >= 2)
        def _():
            pltpu.make_async_copy(st0, dst(n_blocks - 2), sout0).wait()
            pltpu.make_async_copy(st1, dst(n_blocks - 2), sout1).wait()

        @pl.when(n_blocks == 1)
        def _():
            pltpu.make_async_copy(st0, dst(0), sout0).wait()

        if tail:
            @pl.when(wid == _NUM_WORKERS - 1)
            def _():
                pltpu.sync_copy(tail1d, tailv)
                pltpu.sync_copy(
                    tailv, lin.at[pl.ds(full_blocks * block_elems, tail * dim)]
                )

    return detile


@functools.cache
def _make_gather(num_rows, dim, batch, n_fields):
    assert batch == 128 * _NUM_WORKERS and dim % 8 == 0
    rows = batch * n_fields
    r_per_w = rows // _NUM_WORKERS
    ntr = dim // 8
    n_tiles = n_fields * ntr
    chunk_rows = _LANES * n_fields
    n_chunks = r_per_w // chunk_rows

    @functools.partial(
        pl.kernel,
        mesh=_mesh(),
        out_type=jax.ShapeDtypeStruct((n_tiles, _NUM_WORKERS, 1024),
                                      jnp.float32),
        scratch_types=[
            pltpu.VMEM((r_per_w,), jnp.int32),
            pltpu.VMEM((chunk_rows, dim), jnp.float32),
            pltpu.VMEM((n_tiles, 1, 1024), jnp.float32),
            pltpu.SemaphoreType.DMA,
        ],
        compiler_params=pltpu.CompilerParams(
            use_tc_tiling_on_sc=False, needs_layout_passes=False
        ),
    )
    def gather(idx_hbm, table_hbm, out_hbm, idx_v, rows_v, stage, sem):
        wid = lax.axis_index("s") * _NUM_CORES + lax.axis_index("c")
        pltpu.sync_copy(idx_hbm.at[pl.ds(wid * r_per_w, r_per_w)], idx_v)
        lanes = lax.iota(jnp.int32, _LANES)
        lanes_nf = lanes * n_fields
        rvs = [(lanes + r0) % 8 for r0 in range(8)]
        rvs128 = [rv * 128 for rv in rvs]
        zero = lanes * 0

        for cc in range(n_chunks):
            pltpu.async_copy(
                table_hbm.at[idx_v.at[pl.ds(cc * chunk_rows, chunk_rows)]],
                rows_v, sem,
            ).wait()
            lt = lanes + (cc * _LANES)

            @plsc.parallel_loop(0, n_tiles, unroll=2)
            def _(t):
                f = t // ntr
                tr = t % ntr
                tv = zero + t
                rowv = lanes_nf + f
                for r0 in range(8):
                    x = plsc.load_gather(rows_v, [rowv, rvs[r0] + tr * 8])
                    plsc.store_scatter(stage, [tv, zero, rvs128[r0] + lt], x)

        pltpu.sync_copy(stage, out_hbm.at[:, pl.ds(wid, 1), :])

    return gather


def _kernel_impl(inputs, table):
    batch, n_fields = inputs.shape
    num_rows, dim = table.shape
    idx = inputs.reshape(-1).astype(jnp.int32)
    tail_start = (num_rows // _SB) * _SB
    tail1d = lax.slice(table, (tail_start, 0), (num_rows, dim)).reshape(-1)
    lin = _make_detile(num_rows, dim)(table.T, tail1d)
    out3 = _make_gather(num_rows, dim, batch, n_fields)(
        idx, lin.reshape(num_rows, dim)
    )
    out5 = out3.reshape(n_fields, dim // 8, _NUM_WORKERS, 8, 128)
    return jnp.transpose(out5, (2, 4, 0, 1, 3)).reshape(batch, n_fields, dim)


kernel = jax.jit(_kernel_impl)
